# baseline (speedup 1.0000x reference)
import jax
import jax.numpy as jnp
from jax import lax
from jax.experimental import pallas as pl
from jax.experimental.pallas import tpu as pltpu
from jax.experimental.pallas import tpu_sc as plsc

N_VENDORS = 1000000
N_WEEKS = 520
EMB = 16
B = 16384
NW = 32
BPW = B // NW
NCHUNK = 4
CHUNK = BPW // NCHUNK
NBLK = BPW // EMB


def _fe_kernel(vid_hbm, wk_hbm, lc_hbm, vtab_hbm, wtab_hbm, wv_hbm, ww_hbm,
               misc_hbm, out_hbm, idx_v, wk_v, lc_v, vrows, wtab_v, wv_r,
               ww_r, misc_r, out_v, sem):
    nc = 2
    wid = lax.axis_index("s") * nc + lax.axis_index("c")

    pltpu.sync_copy(vid_hbm.at[pl.ds(wid * NCHUNK, NCHUNK)], idx_v)
    copies = []
    for j in range(NCHUNK):
        copies.append(pltpu.async_copy(
            vtab_hbm.at[idx_v.at[j]], vrows.at[pl.ds(j * CHUNK, CHUNK)], sem))
    pltpu.sync_copy(wk_hbm.at[pl.ds(wid * NBLK, NBLK)], wk_v)
    pltpu.sync_copy(lc_hbm.at[pl.ds(wid * NBLK, NBLK)], lc_v)
    pltpu.sync_copy(wtab_hbm, wtab_v)
    pltpu.sync_copy(wv_hbm, wv_r)
    pltpu.sync_copy(ww_hbm, ww_r)
    pltpu.sync_copy(misc_hbm, misc_r)

    iota = lax.iota(jnp.int32, EMB)
    zero = jnp.zeros((EMB,), jnp.int32)
    scale = misc_r[0] * misc_r[1]
    bias = misc_r[2]

    for c in copies:
        c.wait()

    def block(t, carry):
        wid_vec = wk_v[t]
        row0 = t * EMB + iota
        acc0 = lc_v[t] * scale + bias
        acc1 = jnp.zeros((EMB,), jnp.float32)
        acc2 = jnp.zeros((EMB,), jnp.float32)
        acc3 = jnp.zeros((EMB,), jnp.float32)
        accs = [acc0, acc1, acc2, acc3]
        for k in range(EMB):
            kf = zero + k
            a = accs[k % 4]
            a = a + plsc.load_gather(vrows, [row0, kf]) * wv_r[k]
            a = a + plsc.load_gather(wtab_v, [wid_vec, kf]) * ww_r[k]
            accs[k % 4] = a
        out_v[t] = (accs[0] + accs[1]) + (accs[2] + accs[3])
        return carry

    lax.fori_loop(0, NBLK, block, 0)
    pltpu.sync_copy(out_v, out_hbm.at[pl.ds(wid * NBLK, NBLK)])


@jax.jit
def kernel(vendor_ids, week_ids, log_clicks, vendor_emb, week_emb, click_w,
           comb_W, comb_b):
    vid = vendor_ids.reshape(NW * NCHUNK, CHUNK)
    wk = week_ids.reshape(NW * NBLK, EMB)
    lc = log_clicks.reshape(NW * NBLK, EMB)
    wv = jnp.broadcast_to(comb_W[0, 0:EMB, None], (EMB, EMB))
    ww = jnp.broadcast_to(comb_W[0, EMB:2 * EMB, None], (EMB, EMB))
    misc = jnp.broadcast_to(
        jnp.concatenate([comb_W[0, 2 * EMB:], click_w[0], comb_b])[:, None],
        (3, EMB))

    mesh = plsc.VectorSubcoreMesh(core_axis_name="c", subcore_axis_name="s")
    run = pl.kernel(
        _fe_kernel, mesh=mesh,
        compiler_params=pltpu.CompilerParams(
            needs_layout_passes=False, use_tc_tiling_on_sc=False),
        out_type=jax.ShapeDtypeStruct((NW * NBLK, EMB), jnp.float32),
        scratch_types=[
            pltpu.VMEM((NCHUNK, CHUNK), jnp.int32),
            pltpu.VMEM((NBLK, EMB), jnp.int32),
            pltpu.VMEM((NBLK, EMB), jnp.float32),
            pltpu.VMEM((BPW, EMB), jnp.float32),
            pltpu.VMEM((N_WEEKS, EMB), jnp.float32),
            pltpu.VMEM((EMB, EMB), jnp.float32),
            pltpu.VMEM((EMB, EMB), jnp.float32),
            pltpu.VMEM((3, EMB), jnp.float32),
            pltpu.VMEM((NBLK, EMB), jnp.float32),
            pltpu.SemaphoreType.DMA,
        ],
    )
    out = run(vid, wk, lc, vendor_emb, week_emb, wv, ww, misc)
    return out.reshape(B)
